# Initial kernel scaffold; baseline (speedup 1.0000x reference)
import jax
import jax.numpy as jnp
from jax.experimental import pallas as pl


def kernel(X, y, univariate_vars, univariate_weights_0, univariate_weights_1, bivariate_vars_1, bivariate_vars_2, bivariate_weights_00, bivariate_weights_01, bivariate_weights_10, bivariate_weights_11):
    raise NotImplementedError("write your pallas kernel here")


# speedup vs baseline: 1.0346x; 1.0346x over previous
import functools

import jax
import jax.numpy as jnp
from jax import lax
from jax.experimental import pallas as pl
from jax.experimental.pallas import tpu as pltpu
from jax.experimental.pallas import tpu_sc as plsc

NC = 2
NS = 16
L = 16
NW = NC * NS

N_ROWS = 16384
D = 127
RPW = N_ROWS // NW
G = RPW // L
NU = 16
NB = 24


def _sc_kernel(x_hbm, y_hbm, ui_hbm, a_hbm, bi_hbm, bj_hbm, b1_hbm, b2_hbm,
               bb_hbm, c0_hbm, out_hbm,
               xv, yv, accv, uiv, av, biv, bjv, b1v, b2v, bbv, c0v, sem):
    wid = lax.axis_index("s") * NC + lax.axis_index("c")
    base = wid * RPW

    copies = [
        pltpu.async_copy(x_hbm.at[pl.ds(base * D, RPW * D)], xv, sem),
        pltpu.async_copy(y_hbm.at[pl.ds(base, RPW)], yv, sem),
        pltpu.async_copy(ui_hbm, uiv, sem),
        pltpu.async_copy(a_hbm, av, sem),
        pltpu.async_copy(bi_hbm, biv, sem),
        pltpu.async_copy(bj_hbm, bjv, sem),
        pltpu.async_copy(b1_hbm, b1v, sem),
        pltpu.async_copy(b2_hbm, b2v, sem),
        pltpu.async_copy(bb_hbm, bbv, sem),
        pltpu.async_copy(c0_hbm, c0v, sem),
    ]
    for c in copies:
        c.wait()

    def group_body(g, _):
        rowoff = (g * L + lax.iota(jnp.int32, L)) * D
        yg = yv[pl.ds(g * L, L)]
        acc = c0v[...]
        for t in range(NU):
            idxs = uiv[t, :]
            tv = plsc.load_gather(xv, [rowoff + jnp.minimum(idxs, D - 1)])
            tv = jnp.where(idxs == D, yg, tv)
            acc = acc + av[t, :] * tv
        for t in range(NB):
            iis = biv[t, :]
            jjs = bjv[t, :]
            t1 = plsc.load_gather(xv, [rowoff + jnp.minimum(iis, D - 1)])
            t1 = jnp.where(iis == D, yg, t1)
            t2 = plsc.load_gather(xv, [rowoff + jnp.minimum(jjs, D - 1)])
            t2 = jnp.where(jjs == D, yg, t2)
            acc = acc + t1 * (b1v[t, :] + bbv[t, :] * t2) + b2v[t, :] * t2
        accv[pl.ds(g * L, L)] = acc
        return _

    lax.fori_loop(0, G, group_body, None)
    pltpu.sync_copy(accv, out_hbm.at[pl.ds(base, RPW)])


@functools.partial(
    pl.kernel,
    out_type=jax.ShapeDtypeStruct((N_ROWS,), jnp.float32),
    mesh=plsc.VectorSubcoreMesh(core_axis_name="c", subcore_axis_name="s",
                                num_cores=NC, num_subcores=NS),
    scratch_types=[
        pltpu.VMEM((RPW * D,), jnp.float32),
        pltpu.VMEM((RPW,), jnp.float32),
        pltpu.VMEM((RPW,), jnp.float32),
        pltpu.VMEM((NU, L), jnp.int32),
        pltpu.VMEM((NU, L), jnp.float32),
        pltpu.VMEM((NB, L), jnp.int32),
        pltpu.VMEM((NB, L), jnp.int32),
        pltpu.VMEM((NB, L), jnp.float32),
        pltpu.VMEM((NB, L), jnp.float32),
        pltpu.VMEM((NB, L), jnp.float32),
        pltpu.VMEM((L,), jnp.float32),
        pltpu.SemaphoreType.DMA,
    ],
    compiler_params=pltpu.CompilerParams(needs_layout_passes=False),
)
def _pgm_loss_sc(x_hbm, y_hbm, ui_hbm, a_hbm, bi_hbm, bj_hbm, b1_hbm, b2_hbm,
                 bb_hbm, c0_hbm, out_hbm,
                 xv, yv, accv, uiv, av, biv, bjv, b1v, b2v, bbv, c0v, sem):
    _sc_kernel(x_hbm, y_hbm, ui_hbm, a_hbm, bi_hbm, bj_hbm, b1_hbm, b2_hbm,
               bb_hbm, c0_hbm, out_hbm,
               xv, yv, accv, uiv, av, biv, bjv, b1v, b2v, bbv, c0v, sem)


def kernel(X, y, univariate_vars, univariate_weights_0, univariate_weights_1,
           bivariate_vars_1, bivariate_vars_2, bivariate_weights_00,
           bivariate_weights_01, bivariate_weights_10, bivariate_weights_11):
    a = univariate_weights_1 - univariate_weights_0
    b1 = bivariate_weights_10 - bivariate_weights_00
    b2 = bivariate_weights_01 - bivariate_weights_00
    bb = (bivariate_weights_00 - bivariate_weights_01
          - bivariate_weights_10 + bivariate_weights_11)
    c0 = jnp.sum(univariate_weights_0) + jnp.sum(bivariate_weights_00)

    ui_s = jnp.broadcast_to(univariate_vars[:, None], (NU, L))
    a_s = jnp.broadcast_to(a[:, None], (NU, L))
    bi_s = jnp.broadcast_to(bivariate_vars_1[:, None], (NB, L))
    bj_s = jnp.broadcast_to(bivariate_vars_2[:, None], (NB, L))
    b1_s = jnp.broadcast_to(b1[:, None], (NB, L))
    b2_s = jnp.broadcast_to(b2[:, None], (NB, L))
    bb_s = jnp.broadcast_to(bb[:, None], (NB, L))
    c0_s = jnp.full((L,), c0, dtype=jnp.float32)

    return _pgm_loss_sc(X.reshape(-1), y[:, 0], ui_s, a_s, bi_s, bj_s, b1_s,
                        b2_s, bb_s, c0_s)


# speedup vs baseline: 1.3910x; 1.3445x over previous
import functools

import jax
import jax.numpy as jnp
from jax import lax
from jax.experimental import pallas as pl
from jax.experimental.pallas import tpu as pltpu
from jax.experimental.pallas import tpu_sc as plsc

NC = 2
NS = 16
L = 16
NW = NC * NS

N_ROWS = 16384
D = 127
RPW = N_ROWS // NW
G = RPW // L
U = 4
NU = 16
NB = 24


def _splat(vec, i):
    return vec.at[jnp.full((L,), i, dtype=jnp.int32)].get(
        mode="promise_in_bounds")


def _sc_body(x_hbm, y_hbm, uv_hbm, w0_hbm, w1_hbm, bv1_hbm, bv2_hbm,
             w00_hbm, w01_hbm, w10_hbm, w11_hbm, out_hbm,
             xv, yv, accv, uvv, w0v, w1v, bv1v, bv2v, w00v, w01v, w10v,
             w11v, sem):
    wid = lax.axis_index("s") * NC + lax.axis_index("c")
    base = wid * RPW

    copies = [
        pltpu.async_copy(x_hbm.at[pl.ds(base * D, RPW * D)], xv, sem),
        pltpu.async_copy(y_hbm.at[pl.ds(base, RPW)], yv, sem),
        pltpu.async_copy(uv_hbm, uvv, sem),
        pltpu.async_copy(w0_hbm, w0v, sem),
        pltpu.async_copy(w1_hbm, w1v, sem),
        pltpu.async_copy(bv1_hbm, bv1v, sem),
        pltpu.async_copy(bv2_hbm, bv2v, sem),
        pltpu.async_copy(w00_hbm, w00v, sem),
        pltpu.async_copy(w01_hbm, w01v, sem),
        pltpu.async_copy(w10_hbm, w10v, sem),
        pltpu.async_copy(w11_hbm, w11v, sem),
    ]
    for c in copies:
        c.wait()

    lane = lax.iota(jnp.int32, L)

    uvec = uvv[...]
    avec = w1v[...] - w0v[...]

    def chunks(ref):
        return ref[pl.ds(0, L)], ref[pl.ds(8, L)]

    i1, i2 = chunks(bv1v)
    j1, j2 = chunks(bv2v)
    w00a, w00b = chunks(w00v)
    w01a, w01b = chunks(w01v)
    w10a, w10b = chunks(w10v)
    w11a, w11b = chunks(w11v)
    b1a, b1b = w10a - w00a, w10b - w00b
    b2a, b2b = w01a - w00a, w01b - w00b
    bba, bbb = w00a - w01a - w10a + w11a, w00b - w01b - w10b + w11b

    zeros = jnp.zeros((L,), jnp.float32)
    c0_parts = (w0v[...] + w00a + jnp.where(lane >= 8, w00b, zeros))
    c0 = jnp.sum(c0_parts)
    c0v = jnp.full((L,), c0, dtype=jnp.float32)

    def iter_body(it, _):
        g0 = it * U
        rowoffs = [(g0 + u) * (L * D) + lane * D for u in range(U)]
        ygs = [yv[pl.ds((g0 + u) * L, L)] for u in range(U)]
        accs = [c0v for _ in range(U)]

        for t in range(NU):
            idxs = _splat(uvec, t)
            cl = jnp.minimum(idxs, D - 1)
            isy = idxs == D
            at = _splat(avec, t)
            for u in range(U):
                tv = plsc.load_gather(xv, [rowoffs[u] + cl])
                tv = jnp.where(isy, ygs[u], tv)
                accs[u] = accs[u] + at * tv

        for t in range(NB):
            if t < 8:
                iv, jv = i1, j1
                b1, b2, bb = b1a, b2a, bba
                e = t
            else:
                iv, jv = i2, j2
                b1, b2, bb = b1b, b2b, bbb
                e = t - 8
            iis = _splat(iv, e)
            jjs = _splat(jv, e)
            cli = jnp.minimum(iis, D - 1)
            clj = jnp.minimum(jjs, D - 1)
            isyi = iis == D
            isyj = jjs == D
            b1t = _splat(b1, e)
            b2t = _splat(b2, e)
            bbt = _splat(bb, e)
            for u in range(U):
                t1 = plsc.load_gather(xv, [rowoffs[u] + cli])
                t1 = jnp.where(isyi, ygs[u], t1)
                t2 = plsc.load_gather(xv, [rowoffs[u] + clj])
                t2 = jnp.where(isyj, ygs[u], t2)
                accs[u] = accs[u] + t1 * (b1t + bbt * t2) + b2t * t2

        for u in range(U):
            accv[pl.ds((g0 + u) * L, L)] = accs[u]
        return _

    lax.fori_loop(0, G // U, iter_body, None)
    pltpu.sync_copy(accv, out_hbm.at[pl.ds(base, RPW)])


@functools.partial(
    pl.kernel,
    out_type=jax.ShapeDtypeStruct((N_ROWS,), jnp.float32),
    mesh=plsc.VectorSubcoreMesh(core_axis_name="c", subcore_axis_name="s",
                                num_cores=NC, num_subcores=NS),
    scratch_types=[
        pltpu.VMEM((RPW * D,), jnp.float32),
        pltpu.VMEM((RPW,), jnp.float32),
        pltpu.VMEM((RPW,), jnp.float32),
        pltpu.VMEM((NU,), jnp.int32),
        pltpu.VMEM((NU,), jnp.float32),
        pltpu.VMEM((NU,), jnp.float32),
        pltpu.VMEM((NB,), jnp.int32),
        pltpu.VMEM((NB,), jnp.int32),
        pltpu.VMEM((NB,), jnp.float32),
        pltpu.VMEM((NB,), jnp.float32),
        pltpu.VMEM((NB,), jnp.float32),
        pltpu.VMEM((NB,), jnp.float32),
        pltpu.SemaphoreType.DMA,
    ],
    compiler_params=pltpu.CompilerParams(needs_layout_passes=False),
)
def _pgm_loss_sc(x_hbm, y_hbm, uv_hbm, w0_hbm, w1_hbm, bv1_hbm, bv2_hbm,
                 w00_hbm, w01_hbm, w10_hbm, w11_hbm, out_hbm,
                 xv, yv, accv, uvv, w0v, w1v, bv1v, bv2v, w00v, w01v, w10v,
                 w11v, sem):
    _sc_body(x_hbm, y_hbm, uv_hbm, w0_hbm, w1_hbm, bv1_hbm, bv2_hbm,
             w00_hbm, w01_hbm, w10_hbm, w11_hbm, out_hbm,
             xv, yv, accv, uvv, w0v, w1v, bv1v, bv2v, w00v, w01v, w10v,
             w11v, sem)


def kernel(X, y, univariate_vars, univariate_weights_0, univariate_weights_1,
           bivariate_vars_1, bivariate_vars_2, bivariate_weights_00,
           bivariate_weights_01, bivariate_weights_10, bivariate_weights_11):
    return _pgm_loss_sc(X.reshape(-1), y[:, 0], univariate_vars,
                        univariate_weights_0, univariate_weights_1,
                        bivariate_vars_1, bivariate_vars_2,
                        bivariate_weights_00, bivariate_weights_01,
                        bivariate_weights_10, bivariate_weights_11)
